# baseline (speedup 1.0000x reference)
import functools

import numpy as np
import jax
import jax.numpy as jnp
from jax.experimental import pallas as pl
from jax.experimental.pallas import tpu as pltpu


def _se_kernel(x_ref, w_enc_ref, w_dec_ref, o_ref, acc_ref, *, inv_s, ht):
    t = pl.program_id(1)
    nt = pl.num_programs(1)

    @pl.when(t == 0)
    def _():
        acc_ref[...] = jnp.zeros_like(acc_ref)

    xt = x_ref[0].astype(jnp.float32)
    o_ref[0, :, pl.ds(t * ht, ht), :] = xt.astype(o_ref.dtype)
    acc_ref[...] += jnp.sum(jnp.sum(xt, axis=2), axis=1, keepdims=True)

    @pl.when(t == nt - 1)
    def _():
        mean = acc_ref[...] * inv_s
        z = jnp.dot(w_enc_ref[...], mean, preferred_element_type=jnp.float32)
        z = jnp.maximum(z, 0.0)
        a = jnp.dot(w_dec_ref[...], z, preferred_element_type=jnp.float32)
        a = 1.0 / (1.0 + jnp.exp(-a))
        att = a[:, :, None].astype(o_ref.dtype)
        o_ref[0] = o_ref[0] * att


def kernel(x, w_enc, w_dec):
    N, C, H, W = (int(d) for d in x.shape)
    Cr = int(w_enc.shape[0])
    S = H * W

    HT = 16 if H % 16 == 0 else H
    nt = H // HT

    y = pl.pallas_call(
        functools.partial(_se_kernel, inv_s=1.0 / float(S), ht=HT),
        out_shape=jax.ShapeDtypeStruct((N, C, H, W), x.dtype),
        grid=(N, nt),
        in_specs=[
            pl.BlockSpec((1, C, HT, W), lambda n, t: (n, 0, t, 0)),
            pl.BlockSpec((Cr, C), lambda n, t: (0, 0)),
            pl.BlockSpec((C, Cr), lambda n, t: (0, 0)),
        ],
        out_specs=pl.BlockSpec((1, C, H, W), lambda n, t: (n, 0, 0, 0)),
        scratch_shapes=[pltpu.VMEM((C, 1), jnp.float32)],
        compiler_params=pltpu.CompilerParams(
            dimension_semantics=("parallel", "arbitrary"),
            vmem_limit_bytes=58 * 1024 * 1024,
        ),
    )(x, w_enc, w_dec)

    return y


# speedup vs baseline: 7.2268x; 7.2268x over previous
import functools

import numpy as np
import jax
import jax.numpy as jnp
from jax.experimental import pallas as pl
from jax.experimental.pallas import tpu as pltpu


def _se_kernel(x_ref, w_enc_t_ref, w_dec_t_ref, o_ref, *, inv_s):
    x = x_ref[0].astype(jnp.float32)
    hw = x.shape[0] * x.shape[1]
    xm = x.reshape(hw, x.shape[2])
    mean = jnp.sum(xm, axis=0, keepdims=True) * inv_s
    z = jnp.dot(mean, w_enc_t_ref[...], preferred_element_type=jnp.float32)
    z = jnp.maximum(z, 0.0)
    a = jnp.dot(z, w_dec_t_ref[...], preferred_element_type=jnp.float32)
    a = 1.0 / (1.0 + jnp.exp(-a))
    o_ref[0] = (x * a[None]).astype(o_ref.dtype)


def kernel(x, w_enc, w_dec):
    N, C, H, W = (int(d) for d in x.shape)
    Cr = int(w_enc.shape[0])
    S = H * W

    xt = jnp.transpose(x, (0, 2, 3, 1))

    yt = pl.pallas_call(
        functools.partial(_se_kernel, inv_s=1.0 / float(S)),
        out_shape=jax.ShapeDtypeStruct((N, H, W, C), x.dtype),
        grid=(N,),
        in_specs=[
            pl.BlockSpec((1, H, W, C), lambda n: (n, 0, 0, 0)),
            pl.BlockSpec((C, Cr), lambda n: (0, 0)),
            pl.BlockSpec((Cr, C), lambda n: (0, 0)),
        ],
        out_specs=pl.BlockSpec((1, H, W, C), lambda n: (n, 0, 0, 0)),
        compiler_params=pltpu.CompilerParams(
            dimension_semantics=("parallel",),
            vmem_limit_bytes=48 * 1024 * 1024,
        ),
    )(xt, w_enc.T, w_dec.T)

    return jnp.transpose(yt, (0, 3, 1, 2))


# speedup vs baseline: 7.2286x; 1.0002x over previous
import functools

import numpy as np
import jax
import jax.numpy as jnp
from jax.experimental import pallas as pl
from jax.experimental.pallas import tpu as pltpu


def _se_kernel(x_ref, w_enc_ref, w_dec_ref, o_ref, *, inv_s):
    x = x_ref[0].astype(jnp.float32)
    hw = x.shape[0] * x.shape[1]
    xm = x.reshape(hw, x.shape[2])
    mean = jnp.sum(xm, axis=0, keepdims=True) * inv_s
    z = jax.lax.dot_general(mean, w_enc_ref[...], (((1,), (1,)), ((), ())),
                            preferred_element_type=jnp.float32)
    z = jnp.maximum(z, 0.0)
    a = jax.lax.dot_general(z, w_dec_ref[...], (((1,), (1,)), ((), ())),
                            preferred_element_type=jnp.float32)
    a = 1.0 / (1.0 + jnp.exp(-a))
    o_ref[0] = (x * a[None]).astype(o_ref.dtype)


def kernel(x, w_enc, w_dec):
    N, C, H, W = (int(d) for d in x.shape)
    Cr = int(w_enc.shape[0])
    S = H * W

    xt = jnp.transpose(x, (0, 2, 3, 1))

    yt = pl.pallas_call(
        functools.partial(_se_kernel, inv_s=1.0 / float(S)),
        out_shape=jax.ShapeDtypeStruct((N, H, W, C), x.dtype),
        grid=(N,),
        in_specs=[
            pl.BlockSpec((1, H, W, C), lambda n: (n, 0, 0, 0)),
            pl.BlockSpec((Cr, C), lambda n: (0, 0)),
            pl.BlockSpec((C, Cr), lambda n: (0, 0)),
        ],
        out_specs=pl.BlockSpec((1, H, W, C), lambda n: (n, 0, 0, 0)),
        compiler_params=pltpu.CompilerParams(
            dimension_semantics=("parallel",),
            vmem_limit_bytes=48 * 1024 * 1024,
        ),
    )(xt, w_enc, w_dec)

    return jnp.transpose(yt, (0, 3, 1, 2))
